# Initial kernel scaffold; baseline (speedup 1.0000x reference)
import jax
import jax.numpy as jnp
from jax.experimental import pallas as pl


def kernel(V, E, X, W1, b1, W2, b2):
    raise NotImplementedError("write your pallas kernel here")


# speedup vs baseline: 51.8193x; 51.8193x over previous
import functools

import jax
import jax.numpy as jnp
from jax import lax
from jax.experimental import pallas as pl
from jax.experimental.pallas import tpu as pltpu
from jax.experimental.pallas import tpu_sc as plsc

N_NODES = 10000
N_EDGES = 320000
D_IN = 128
D_HID = 16
D_OUT = 64

NC = 2
NS = 16
NW = NC * NS

NP = 10240
RPT = NP // NS
EPT = N_EDGES // NW
CH = 2000
NCHUNK = EPT // CH

_MESH = plsc.VectorSubcoreMesh(
    core_axis_name="c", subcore_axis_name="s", num_cores=NC, num_subcores=NS
)


def _sc_deg_body(dst_hbm, outd, didx, ones_v, zb, sdeg):
    c = lax.axis_index("c")
    s = lax.axis_index("s")
    w = c * NS + s

    def fill(i, _):
        ones_v[pl.ds(i * 16, 16)] = jnp.full((16,), 1.0, jnp.float32)
        return 0

    lax.fori_loop(0, CH // 16, fill, 0)

    def zfill(i, _):
        zb[pl.ds(i * 16, 16)] = jnp.zeros((16,), jnp.float32)
        return 0

    lax.fori_loop(0, RPT // 16, zfill, 0)
    pltpu.sync_copy(zb, sdeg.at[pl.ds(s * RPT, RPT)])
    plsc.subcore_barrier()

    def chunk(k, _):
        off = pl.multiple_of(w * EPT + k * CH, 8)
        pltpu.sync_copy(dst_hbm.at[pl.ds(off, CH)], didx)
        pltpu.sync_copy(ones_v, sdeg.at[didx], add=True)
        return 0

    lax.fori_loop(0, NCHUNK, chunk, 0)
    plsc.subcore_barrier()
    pltpu.sync_copy(sdeg.at[pl.ds(s * RPT, RPT)], outd.at[c, pl.ds(s * RPT, RPT)])


def _sc_edge_body(src_hbm, dst_hbm, z_hbm, out, sidx, didx, rows, zb, sacc, sem):
    c = lax.axis_index("c")
    s = lax.axis_index("s")
    w = c * NS + s

    def zfill(i, _):
        zb[i, :] = jnp.zeros((16,), jnp.float32)
        return 0

    lax.fori_loop(0, RPT, zfill, 0)
    pltpu.sync_copy(zb, sacc.at[pl.ds(s * RPT, RPT)])
    plsc.subcore_barrier()

    def chunk(k, _):
        off = pl.multiple_of(w * EPT + k * CH, 8)
        pltpu.sync_copy(src_hbm.at[pl.ds(off, CH)], sidx)
        pltpu.sync_copy(dst_hbm.at[pl.ds(off, CH)], didx)
        pltpu.async_copy(z_hbm.at[sidx], rows, sem).wait()
        pltpu.sync_copy(rows, sacc.at[didx], add=True)
        return 0

    lax.fori_loop(0, NCHUNK, chunk, 0)
    plsc.subcore_barrier()
    pltpu.sync_copy(sacc.at[pl.ds(s * RPT, RPT)], out.at[c, pl.ds(s * RPT, RPT)])


_sc_deg = pl.kernel(
    _sc_deg_body,
    out_type=jax.ShapeDtypeStruct((NC, NP), jnp.float32),
    mesh=_MESH,
    scratch_types=[
        pltpu.VMEM((CH,), jnp.int32),
        pltpu.VMEM((CH,), jnp.float32),
        pltpu.VMEM((RPT,), jnp.float32),
        pltpu.VMEM_SHARED((NP,), jnp.float32),
    ],
)

_sc_edge = pl.kernel(
    _sc_edge_body,
    out_type=jax.ShapeDtypeStruct((NC, NP, D_HID), jnp.float32),
    mesh=_MESH,
    scratch_types=[
        pltpu.VMEM((CH,), jnp.int32),
        pltpu.VMEM((CH,), jnp.int32),
        pltpu.VMEM((CH, D_HID), jnp.float32),
        pltpu.VMEM((RPT, D_HID), jnp.float32),
        pltpu.VMEM_SHARED((NP, D_HID), jnp.float32),
        pltpu.SemaphoreType.DMA,
    ],
    compiler_params=pltpu.CompilerParams(use_tc_tiling_on_sc=False),
)


def _tc_mm1_body(x_ref, w_ref, y_ref):
    y_ref[...] = jnp.dot(x_ref[...], w_ref[...], preferred_element_type=jnp.float32)


def _tc_prep_body(d0_ref, d1_ref, y_ref, dinv_ref, z1_ref):
    deg = d0_ref[...] + d1_ref[...] + 1.0
    dinv = lax.rsqrt(jnp.maximum(deg, 1e-12))
    dinvb = jnp.broadcast_to(dinv, (NP, D_HID))
    dinv_ref[...] = dinvb
    z1_ref[...] = y_ref[...] * dinvb


def _tc_mid_body(p0_ref, p1_ref, z1_ref, dinv_ref, b1_ref, z2_ref):
    dinv = dinv_ref[...]
    pre = dinv * (p0_ref[...] + p1_ref[...] + z1_ref[...]) + b1_ref[...]
    z2_ref[...] = dinv * jnp.maximum(pre, 0.0)


def _tc_out_body(p0_ref, p1_ref, z2_ref, dinv_ref, w2_ref, b2_ref, o_ref):
    a2 = dinv_ref[...] * (p0_ref[...] + p1_ref[...] + z2_ref[...])
    o_ref[...] = (
        jnp.dot(a2, w2_ref[...], preferred_element_type=jnp.float32) + b2_ref[...]
    )


_tc_mm1 = pl.pallas_call(
    _tc_mm1_body, out_shape=jax.ShapeDtypeStruct((NP, D_HID), jnp.float32)
)
_tc_prep = pl.pallas_call(
    _tc_prep_body,
    out_shape=[
        jax.ShapeDtypeStruct((NP, D_HID), jnp.float32),
        jax.ShapeDtypeStruct((NP, D_HID), jnp.float32),
    ],
)
_tc_mid = pl.pallas_call(
    _tc_mid_body, out_shape=jax.ShapeDtypeStruct((NP, D_HID), jnp.float32)
)
_tc_out = pl.pallas_call(
    _tc_out_body, out_shape=jax.ShapeDtypeStruct((NP, D_OUT), jnp.float32)
)


@jax.jit
def kernel(V, E, X, W1, b1, W2, b2):
    src = E[0]
    dst = E[1]
    Xp = jnp.pad(X, ((0, NP - N_NODES), (0, 0)))

    degp = _sc_deg(dst)
    Y = _tc_mm1(Xp, W1)
    dinvb, Z1 = _tc_prep(
        degp[0].reshape(NP, 1), degp[1].reshape(NP, 1), Y
    )
    P1 = _sc_edge(src, dst, Z1)
    Z2 = _tc_mid(P1[0], P1[1], Z1, dinvb, b1.reshape(1, D_HID))
    P2 = _sc_edge(src, dst, Z2)
    O = _tc_out(P2[0], P2[1], Z2, dinvb, W2, b2.reshape(1, D_OUT))
    return O[:N_NODES]


# speedup vs baseline: 65.5577x; 1.2651x over previous
import jax
import jax.numpy as jnp
from jax import lax
from jax.experimental import pallas as pl
from jax.experimental.pallas import tpu as pltpu
from jax.experimental.pallas import tpu_sc as plsc

N_NODES = 10000
N_EDGES = 320000
D_IN = 128
D_HID = 16
D_OUT = 64

NC = 2
NS = 16
NW = NC * NS

NP = 10240
RPT = NP // NS
EPT = N_EDGES // NW
CH = 2000
NCHUNK = EPT // CH

_MESH = plsc.VectorSubcoreMesh(
    core_axis_name="c", subcore_axis_name="s", num_cores=NC, num_subcores=NS
)


def _sc_deg_body(dst_hbm, outd, didx, ones_v, zb, degv, degb_v, sdeg, sem):
    c = lax.axis_index("c")
    s = lax.axis_index("s")
    w = c * NS + s

    idx_cp = pltpu.async_copy(
        dst_hbm.at[pl.ds(w * NCHUNK, NCHUNK)], didx, sem
    )

    def fill(i, _):
        ones_v[pl.ds(i * 16, 16)] = jnp.full((16,), 1.0, jnp.float32)
        return 0

    lax.fori_loop(0, CH // 16, fill, 0)

    def zfill(i, _):
        zb[pl.ds(i * 16, 16)] = jnp.zeros((16,), jnp.float32)
        return 0

    lax.fori_loop(0, RPT // 16, zfill, 0)
    pltpu.sync_copy(zb, sdeg.at[pl.ds(s * RPT, RPT)])
    idx_cp.wait()
    plsc.subcore_barrier()

    scats = []
    for k in range(NCHUNK):
        scats.append(
            pltpu.async_copy(ones_v, sdeg.at[didx.at[k]], sem, add=True)
        )
    for cp in scats:
        cp.wait()
    plsc.subcore_barrier()

    pltpu.sync_copy(sdeg.at[pl.ds(s * RPT, RPT)], degv)

    def bcast(i, _):
        degb_v[pl.ds(i * 16, 16)] = plsc.load_gather(
            degv, [jnp.full((16,), i, jnp.int32)]
        )
        return 0

    lax.fori_loop(0, RPT, bcast, 0)
    pltpu.sync_copy(degb_v, outd.at[c, pl.ds(s * RPT * D_HID, RPT * D_HID)])


def _sc_edge_body(src_hbm, dst_hbm, z_hbm, out, sidx, didx, rows_a, rows_b, zb, sacc, gsem, ssem):
    c = lax.axis_index("c")
    s = lax.axis_index("s")
    w = c * NS + s

    si_cp = pltpu.async_copy(src_hbm.at[pl.ds(w * NCHUNK, NCHUNK)], sidx, gsem.at[0])
    di_cp = pltpu.async_copy(dst_hbm.at[pl.ds(w * NCHUNK, NCHUNK)], didx, gsem.at[1])

    def zfill(i, _):
        zb[i, :] = jnp.zeros((16,), jnp.float32)
        return 0

    lax.fori_loop(0, RPT, zfill, 0)
    pltpu.sync_copy(zb, sacc.at[pl.ds(s * RPT, RPT)])
    si_cp.wait()
    di_cp.wait()
    plsc.subcore_barrier()

    rows = (rows_a, rows_b)
    gathers = [None, None]
    scats = [None, None]
    gathers[0] = pltpu.async_copy(z_hbm.at[sidx.at[0]], rows[0], gsem.at[0])
    for k in range(NCHUNK):
        cur = k % 2
        nxt = 1 - cur
        gathers[cur].wait()
        if k + 1 < NCHUNK:
            if scats[nxt] is not None:
                scats[nxt].wait()
            gathers[nxt] = pltpu.async_copy(
                z_hbm.at[sidx.at[k + 1]], rows[nxt], gsem.at[nxt]
            )
        scats[cur] = pltpu.async_copy(
            rows[cur], sacc.at[didx.at[k]], ssem.at[cur], add=True
        )
    for cp in scats:
        if cp is not None:
            cp.wait()
    plsc.subcore_barrier()
    pltpu.sync_copy(sacc.at[pl.ds(s * RPT, RPT)], out.at[c, pl.ds(s * RPT, RPT)])


_sc_deg = pl.kernel(
    _sc_deg_body,
    out_type=jax.ShapeDtypeStruct((NC, NP * D_HID), jnp.float32),
    mesh=_MESH,
    scratch_types=[
        pltpu.VMEM((NCHUNK, CH), jnp.int32),
        pltpu.VMEM((CH,), jnp.float32),
        pltpu.VMEM((RPT,), jnp.float32),
        pltpu.VMEM((RPT,), jnp.float32),
        pltpu.VMEM((RPT * D_HID,), jnp.float32),
        pltpu.VMEM_SHARED((NP,), jnp.float32),
        pltpu.SemaphoreType.DMA,
    ],
    compiler_params=pltpu.CompilerParams(
        use_tc_tiling_on_sc=False, needs_layout_passes=False
    ),
)

_sc_edge = pl.kernel(
    _sc_edge_body,
    out_type=jax.ShapeDtypeStruct((NC, NP, D_HID), jnp.float32),
    mesh=_MESH,
    scratch_types=[
        pltpu.VMEM((NCHUNK, CH), jnp.int32),
        pltpu.VMEM((NCHUNK, CH), jnp.int32),
        pltpu.VMEM((CH, D_HID), jnp.float32),
        pltpu.VMEM((CH, D_HID), jnp.float32),
        pltpu.VMEM((RPT, D_HID), jnp.float32),
        pltpu.VMEM_SHARED((NP, D_HID), jnp.float32),
        pltpu.SemaphoreType.DMA((2,)),
        pltpu.SemaphoreType.DMA((2,)),
    ],
    compiler_params=pltpu.CompilerParams(use_tc_tiling_on_sc=False),
)


def _tc_prep_body(x_ref, w1_ref, degb_ref, dinv_ref, z1_ref):
    y = jnp.dot(x_ref[...], w1_ref[...], preferred_element_type=jnp.float32)
    deg = degb_ref[0, :N_NODES, :] + degb_ref[1, :N_NODES, :] + 1.0
    dinvb = lax.rsqrt(jnp.maximum(deg, 1e-12))
    dinv_ref[...] = dinvb
    z1_ref[...] = y * dinvb


def _tc_mid_body(p_ref, z1_ref, dinv_ref, b1_ref, z2_ref):
    sagg = p_ref[0, :N_NODES, :] + p_ref[1, :N_NODES, :]
    dinvb = dinv_ref[...]
    pre = dinvb * (sagg + z1_ref[...]) + b1_ref[...]
    z2_ref[...] = dinvb * jnp.maximum(pre, 0.0)


def _tc_out_body(p_ref, z2_ref, dinv_ref, w2_ref, b2_ref, o_ref):
    sagg = p_ref[0, :N_NODES, :] + p_ref[1, :N_NODES, :]
    a2 = dinv_ref[...] * (sagg + z2_ref[...])
    o_ref[...] = (
        jnp.dot(a2, w2_ref[...], preferred_element_type=jnp.float32) + b2_ref[...]
    )


_tc_prep = pl.pallas_call(
    _tc_prep_body,
    out_shape=[
        jax.ShapeDtypeStruct((N_NODES, D_HID), jnp.float32),
        jax.ShapeDtypeStruct((N_NODES, D_HID), jnp.float32),
    ],
)
_tc_mid = pl.pallas_call(
    _tc_mid_body, out_shape=jax.ShapeDtypeStruct((N_NODES, D_HID), jnp.float32)
)
_tc_out = pl.pallas_call(
    _tc_out_body, out_shape=jax.ShapeDtypeStruct((N_NODES, D_OUT), jnp.float32)
)


@jax.jit
def kernel(V, E, X, W1, b1, W2, b2):
    src2 = E[0].reshape(NW * NCHUNK, CH)
    dst2 = E[1].reshape(NW * NCHUNK, CH)

    degb = _sc_deg(dst2).reshape(NC, NP, D_HID)
    dinvb, Z1 = _tc_prep(X, W1, degb)
    P1 = _sc_edge(src2, dst2, Z1)
    Z2 = _tc_mid(P1, Z1, dinvb, b1.reshape(1, D_HID))
    P2 = _sc_edge(src2, dst2, Z2)
    O = _tc_out(P2, Z2, dinvb, W2, b2.reshape(1, D_OUT))
    return O


# speedup vs baseline: 76.6009x; 1.1684x over previous
import jax
import jax.numpy as jnp
from jax import lax
from jax.experimental import pallas as pl
from jax.experimental.pallas import tpu as pltpu
from jax.experimental.pallas import tpu_sc as plsc

N_NODES = 10000
N_EDGES = 320000
D_IN = 128
D_HID = 16
D_OUT = 64

NC = 2
NS = 16
NW = NC * NS

NP = 10240
NPF = NP * D_HID
RPT = NP // NS
EPT = N_EDGES // NW
CH = 2000
NCHUNK = EPT // CH

_MESH = plsc.VectorSubcoreMesh(
    core_axis_name="c", subcore_axis_name="s", num_cores=NC, num_subcores=NS
)


def _sc_deg_body(dst_hbm, outd, didx, ones_v, zb, sdeg, sem):
    c = lax.axis_index("c")
    s = lax.axis_index("s")
    w = c * NS + s

    idx_cp = pltpu.async_copy(dst_hbm.at[pl.ds(w * NCHUNK, NCHUNK)], didx, sem)

    def fill(i, _):
        ones_v[pl.ds(i * 16, 16)] = jnp.full((16,), 1.0, jnp.float32)
        return 0

    lax.fori_loop(0, CH // 16, fill, 0)

    def zfill(i, _):
        zb[pl.ds(i * 16, 16)] = jnp.zeros((16,), jnp.float32)
        return 0

    lax.fori_loop(0, RPT // 16, zfill, 0)
    pltpu.sync_copy(zb, sdeg.at[pl.ds(s * RPT, RPT)])
    idx_cp.wait()
    plsc.subcore_barrier()

    scats = []
    for k in range(NCHUNK):
        scats.append(pltpu.async_copy(ones_v, sdeg.at[didx.at[k]], sem, add=True))
    for cp in scats:
        cp.wait()
    plsc.subcore_barrier()
    pltpu.sync_copy(sdeg.at[pl.ds(s * RPT, RPT)], outd.at[c, pl.ds(s * RPT, RPT)])


def _sc_dinv_body(hist_hbm, outd, h0v, h1v, dinvv, dbv):
    c = lax.axis_index("c")
    s = lax.axis_index("s")
    base = s * RPT

    pltpu.sync_copy(hist_hbm.at[0, pl.ds(base, RPT)], h0v)
    pltpu.sync_copy(hist_hbm.at[1, pl.ds(base, RPT)], h1v)

    def newt(g, _):
        d = h0v[pl.ds(g * 16, 16)] + h1v[pl.ds(g * 16, 16)] + 1.0
        i = plsc.bitcast(d, jnp.int32)
        i = jnp.int32(0x5F3759DF) - lax.shift_right_logical(i, 1)
        y = plsc.bitcast(i, jnp.float32)
        for _ in range(3):
            y = y * (1.5 - 0.5 * d * y * y)
        dinvv[pl.ds(g * 16, 16)] = y
        return 0

    lax.fori_loop(0, RPT // 16, newt, 0)

    def splat(i, _):
        dbv[pl.ds(i * 16, 16)] = plsc.load_gather(
            dinvv, [jnp.full((16,), i, jnp.int32)]
        )
        return 0

    lax.fori_loop(0, RPT, splat, 0)
    pltpu.sync_copy(dbv, outd.at[c, pl.ds(base * D_HID, RPT * D_HID)])


def _edge_pipeline(src_hbm, dst_hbm, zc, sacc, sidx, didx, rows, gsem, ssem, c, w):
    si_cp = pltpu.async_copy(src_hbm.at[pl.ds(w * NCHUNK, NCHUNK)], sidx, gsem.at[0])
    di_cp = pltpu.async_copy(dst_hbm.at[pl.ds(w * NCHUNK, NCHUNK)], didx, gsem.at[1])
    return si_cp, di_cp


def _edge_chunks(zc_core, sacc, sidx, didx, rows, gsem, ssem):
    gathers = [None, None]
    scats = [None, None]
    gathers[0] = pltpu.async_copy(zc_core.at[sidx.at[0]], rows[0], gsem.at[0])
    for k in range(NCHUNK):
        cur = k % 2
        nxt = 1 - cur
        gathers[cur].wait()
        if k + 1 < NCHUNK:
            if scats[nxt] is not None:
                scats[nxt].wait()
            gathers[nxt] = pltpu.async_copy(
                zc_core.at[sidx.at[k + 1]], rows[nxt], gsem.at[nxt]
            )
        scats[cur] = pltpu.async_copy(
            rows[cur], sacc.at[didx.at[k]], ssem.at[cur], add=True
        )
    for cp in scats:
        if cp is not None:
            cp.wait()


def _edge1_body(
    src_hbm, dst_hbm, dinvb_hbm, y_hbm, zc, pout,
    sidx, didx, rows_a, rows_b, dv, zv, sacc, gsem, ssem,
):
    c = lax.axis_index("c")
    s = lax.axis_index("s")
    w = c * NS + s
    base = s * RPT

    si_cp, di_cp = _edge_pipeline(
        src_hbm, dst_hbm, zc, sacc, sidx, didx, None, gsem, ssem, c, w
    )
    pltpu.sync_copy(dinvb_hbm.at[c, pl.ds(base, RPT)], dv)
    pltpu.sync_copy(y_hbm.at[pl.ds(base, RPT)], rows_a.at[pl.ds(0, RPT)])
    sel = jnp.where(c == 0, 1.0, 0.0).astype(jnp.float32)

    def prow(i, _):
        z = rows_a[i, :] * dv[i, :]
        zv[i, :] = z
        rows_b[i, :] = z * sel
        return 0

    lax.fori_loop(0, RPT, prow, 0)
    pltpu.sync_copy(zv, zc.at[c, pl.ds(base, RPT)])
    pltpu.sync_copy(rows_b.at[pl.ds(0, RPT)], sacc.at[pl.ds(base, RPT)])
    si_cp.wait()
    di_cp.wait()
    plsc.subcore_barrier()

    _edge_chunks(zc.at[c], sacc, sidx, didx, (rows_a, rows_b), gsem, ssem)
    plsc.subcore_barrier()

    pltpu.sync_copy(sacc.at[pl.ds(base, RPT)], rows_a.at[pl.ds(0, RPT)])

    def erow(i, _):
        rows_b[i, :] = rows_a[i, :] * dv[i, :]
        return 0

    lax.fori_loop(0, RPT, erow, 0)
    pltpu.sync_copy(rows_b.at[pl.ds(0, RPT)], pout.at[c, pl.ds(base, RPT)])


def _edge2_body(
    src_hbm, dst_hbm, dinvb_hbm, p1_hbm, b1_hbm, zc, tout,
    sidx, didx, rows_a, rows_b, dv, zv, b1v, sacc, gsem, ssem,
):
    c = lax.axis_index("c")
    s = lax.axis_index("s")
    w = c * NS + s
    base = s * RPT

    si_cp, di_cp = _edge_pipeline(
        src_hbm, dst_hbm, zc, sacc, sidx, didx, None, gsem, ssem, c, w
    )
    pltpu.sync_copy(dinvb_hbm.at[c, pl.ds(base, RPT)], dv)
    pltpu.sync_copy(p1_hbm.at[0, pl.ds(base, RPT)], rows_a.at[pl.ds(0, RPT)])
    pltpu.sync_copy(p1_hbm.at[1, pl.ds(base, RPT)], rows_b.at[pl.ds(0, RPT)])
    pltpu.sync_copy(b1_hbm, b1v)
    sel = jnp.where(c == 0, 1.0, 0.0).astype(jnp.float32)
    bvec = b1v[...]

    def prow(i, _):
        pre = rows_a[i, :] + rows_b[i, :] + bvec
        z = dv[i, :] * jnp.maximum(pre, 0.0)
        zv[i, :] = z
        rows_b[i, :] = z * sel
        return 0

    lax.fori_loop(0, RPT, prow, 0)
    pltpu.sync_copy(zv, zc.at[c, pl.ds(base, RPT)])
    pltpu.sync_copy(rows_b.at[pl.ds(0, RPT)], sacc.at[pl.ds(base, RPT)])
    si_cp.wait()
    di_cp.wait()
    plsc.subcore_barrier()

    _edge_chunks(zc.at[c], sacc, sidx, didx, (rows_a, rows_b), gsem, ssem)
    plsc.subcore_barrier()

    pltpu.sync_copy(sacc.at[pl.ds(base, RPT)], rows_a.at[pl.ds(0, RPT)])

    def erow(i, _):
        rows_b[i, :] = rows_a[i, :] * dv[i, :]
        return 0

    lax.fori_loop(0, RPT, erow, 0)
    pltpu.sync_copy(rows_b.at[pl.ds(0, RPT)], tout.at[c, pl.ds(base, RPT)])


_SC_PARAMS = pltpu.CompilerParams(use_tc_tiling_on_sc=False)

_sc_deg = pl.kernel(
    _sc_deg_body,
    out_type=jax.ShapeDtypeStruct((NC, NP), jnp.float32),
    mesh=_MESH,
    scratch_types=[
        pltpu.VMEM((NCHUNK, CH), jnp.int32),
        pltpu.VMEM((CH,), jnp.float32),
        pltpu.VMEM((RPT,), jnp.float32),
        pltpu.VMEM_SHARED((NP,), jnp.float32),
        pltpu.SemaphoreType.DMA,
    ],
    compiler_params=_SC_PARAMS,
)

_sc_dinv = pl.kernel(
    _sc_dinv_body,
    out_type=jax.ShapeDtypeStruct((NC, NPF), jnp.float32),
    mesh=_MESH,
    scratch_types=[
        pltpu.VMEM((RPT,), jnp.float32),
        pltpu.VMEM((RPT,), jnp.float32),
        pltpu.VMEM((RPT,), jnp.float32),
        pltpu.VMEM((RPT * D_HID,), jnp.float32),
    ],
    compiler_params=pltpu.CompilerParams(
        use_tc_tiling_on_sc=False, needs_layout_passes=False
    ),
)

_edge_scratch = [
    pltpu.VMEM((NCHUNK, CH), jnp.int32),
    pltpu.VMEM((NCHUNK, CH), jnp.int32),
    pltpu.VMEM((CH, D_HID), jnp.float32),
    pltpu.VMEM((CH, D_HID), jnp.float32),
    pltpu.VMEM((RPT, D_HID), jnp.float32),
    pltpu.VMEM((RPT, D_HID), jnp.float32),
]

_sc_edge1 = pl.kernel(
    _edge1_body,
    out_type=[
        jax.ShapeDtypeStruct((NC, NP, D_HID), jnp.float32),
        jax.ShapeDtypeStruct((NC, NP, D_HID), jnp.float32),
    ],
    mesh=_MESH,
    scratch_types=_edge_scratch
    + [
        pltpu.VMEM_SHARED((NP, D_HID), jnp.float32),
        pltpu.SemaphoreType.DMA((2,)),
        pltpu.SemaphoreType.DMA((2,)),
    ],
    compiler_params=_SC_PARAMS,
)

_sc_edge2 = pl.kernel(
    _edge2_body,
    out_type=[
        jax.ShapeDtypeStruct((NC, NP, D_HID), jnp.float32),
        jax.ShapeDtypeStruct((NC, NP, D_HID), jnp.float32),
    ],
    mesh=_MESH,
    scratch_types=_edge_scratch
    + [
        pltpu.VMEM((D_HID,), jnp.float32),
        pltpu.VMEM_SHARED((NP, D_HID), jnp.float32),
        pltpu.SemaphoreType.DMA((2,)),
        pltpu.SemaphoreType.DMA((2,)),
    ],
    compiler_params=_SC_PARAMS,
)


def _tc_mm1_body(x_ref, w_ref, y_ref):
    y_ref[...] = jnp.dot(x_ref[...], w_ref[...], preferred_element_type=jnp.float32)


def _tc_out_body(t_ref, w2_ref, b2_ref, o_ref):
    a2 = t_ref[0, :N_NODES, :] + t_ref[1, :N_NODES, :]
    o_ref[...] = (
        jnp.dot(a2, w2_ref[...], preferred_element_type=jnp.float32) + b2_ref[...]
    )


_tc_mm1 = pl.pallas_call(
    _tc_mm1_body, out_shape=jax.ShapeDtypeStruct((N_NODES, D_HID), jnp.float32)
)
_tc_out = pl.pallas_call(
    _tc_out_body, out_shape=jax.ShapeDtypeStruct((N_NODES, D_OUT), jnp.float32)
)


@jax.jit
def kernel(V, E, X, W1, b1, W2, b2):
    src2 = E[0].reshape(NW * NCHUNK, CH)
    dst2 = E[1].reshape(NW * NCHUNK, CH)

    hist = _sc_deg(dst2)
    dinvb = _sc_dinv(hist).reshape(NC, NP, D_HID)
    Y = _tc_mm1(X, W1)
    Yp = jnp.pad(Y, ((0, NP - N_NODES), (0, 0)))
    _Z1c, P1 = _sc_edge1(src2, dst2, dinvb, Yp)
    _Z2c, T = _sc_edge2(src2, dst2, dinvb, P1, b1)
    return _tc_out(T, W2, b2.reshape(1, D_OUT))


# speedup vs baseline: 77.8211x; 1.0159x over previous
import jax
import jax.numpy as jnp
from jax import lax
from jax.experimental import pallas as pl
from jax.experimental.pallas import tpu as pltpu
from jax.experimental.pallas import tpu_sc as plsc

N_NODES = 10000
N_EDGES = 320000
D_IN = 128
D_HID = 16
D_OUT = 64

NC = 2
NS = 16
NW = NC * NS

NP = 10240
NPF = NP * D_HID
RPT = NP // NS
EPT = N_EDGES // NW
CH = 2000
NCHUNK = EPT // CH

_MESH = plsc.VectorSubcoreMesh(
    core_axis_name="c", subcore_axis_name="s", num_cores=NC, num_subcores=NS
)


def _sc_deg_body(dst_hbm, outd, didx, ones_v, zb, sdeg, sem):
    c = lax.axis_index("c")
    s = lax.axis_index("s")
    w = c * NS + s

    idx_cp = pltpu.async_copy(dst_hbm.at[pl.ds(w * NCHUNK, NCHUNK)], didx, sem)

    def fill(i, _):
        ones_v[pl.ds(i * 16, 16)] = jnp.full((16,), 1.0, jnp.float32)
        return 0

    lax.fori_loop(0, CH // 16, fill, 0)

    def zfill(i, _):
        zb[pl.ds(i * 16, 16)] = jnp.zeros((16,), jnp.float32)
        return 0

    lax.fori_loop(0, RPT // 16, zfill, 0)
    pltpu.sync_copy(zb, sdeg.at[pl.ds(s * RPT, RPT)])
    idx_cp.wait()
    plsc.subcore_barrier()

    scats = []
    for k in range(NCHUNK):
        scats.append(pltpu.async_copy(ones_v, sdeg.at[didx.at[k]], sem, add=True))
    for cp in scats:
        cp.wait()
    plsc.subcore_barrier()
    pltpu.sync_copy(sdeg.at[pl.ds(s * RPT, RPT)], outd.at[c, pl.ds(s * RPT, RPT)])


def _sc_dinv_body(hist_hbm, outd, h0v, h1v, dinvv, dbv):
    c = lax.axis_index("c")
    s = lax.axis_index("s")
    base = s * RPT

    pltpu.sync_copy(hist_hbm.at[0, pl.ds(base, RPT)], h0v)
    pltpu.sync_copy(hist_hbm.at[1, pl.ds(base, RPT)], h1v)

    def newt(g, _):
        d = h0v[pl.ds(g * 16, 16)] + h1v[pl.ds(g * 16, 16)] + 1.0
        i = plsc.bitcast(d, jnp.int32)
        i = jnp.int32(0x5F3759DF) - lax.shift_right_logical(i, 1)
        y = plsc.bitcast(i, jnp.float32)
        for _ in range(3):
            y = y * (1.5 - 0.5 * d * y * y)
        dinvv[pl.ds(g * 16, 16)] = y
        return 0

    lax.fori_loop(0, RPT // 16, newt, 0)

    def splat(i, _):
        dbv[pl.ds(i * 16, 16)] = plsc.load_gather(
            dinvv, [jnp.full((16,), i, jnp.int32)]
        )
        return 0

    lax.fori_loop(0, RPT, splat, 0)
    pltpu.sync_copy(dbv, outd.at[c, pl.ds(base * D_HID, RPT * D_HID)])


def _edge_pipeline(src_hbm, dst_hbm, zc, sacc, sidx, didx, rows, gsem, ssem, c, w):
    si_cp = pltpu.async_copy(src_hbm.at[pl.ds(w * NCHUNK, NCHUNK)], sidx, gsem.at[0])
    di_cp = pltpu.async_copy(dst_hbm.at[pl.ds(w * NCHUNK, NCHUNK)], didx, gsem.at[1])
    return si_cp, di_cp


def _edge_chunks(zc_core, sacc, sidx, didx, rows, gsem, ssem):
    gathers = [None, None]
    scats = [None, None]
    gathers[0] = pltpu.async_copy(zc_core.at[sidx.at[0]], rows[0], gsem.at[0])
    for k in range(NCHUNK):
        cur = k % 2
        nxt = 1 - cur
        gathers[cur].wait()
        if k + 1 < NCHUNK:
            if scats[nxt] is not None:
                scats[nxt].wait()
            gathers[nxt] = pltpu.async_copy(
                zc_core.at[sidx.at[k + 1]], rows[nxt], gsem.at[nxt]
            )
        scats[cur] = pltpu.async_copy(
            rows[cur], sacc.at[didx.at[k]], ssem.at[cur], add=True
        )
    for cp in scats:
        if cp is not None:
            cp.wait()


def _edge1_body(
    src_hbm, dst_hbm, dinvb_hbm, y_hbm, zc, pout,
    sidx, didx, rows_a, rows_b, dv, zv, sacc, gsem, ssem,
):
    c = lax.axis_index("c")
    s = lax.axis_index("s")
    w = c * NS + s
    base = s * RPT

    si_cp, di_cp = _edge_pipeline(
        src_hbm, dst_hbm, zc, sacc, sidx, didx, None, gsem, ssem, c, w
    )
    pltpu.sync_copy(dinvb_hbm.at[c, pl.ds(base, RPT)], dv)
    pltpu.sync_copy(y_hbm.at[pl.ds(base, RPT)], rows_a.at[pl.ds(0, RPT)])
    sel = jnp.where(c == 0, 1.0, 0.0).astype(jnp.float32)

    def prow(g, _):
        for j in range(8):
            i = g * 8 + j
            z = rows_a[i, :] * dv[i, :]
            zv[i, :] = z
            rows_b[i, :] = z * sel
        return 0

    lax.fori_loop(0, RPT // 8, prow, 0)
    pltpu.sync_copy(zv, zc.at[c, pl.ds(base, RPT)])
    pltpu.sync_copy(rows_b.at[pl.ds(0, RPT)], sacc.at[pl.ds(base, RPT)])
    si_cp.wait()
    di_cp.wait()
    plsc.subcore_barrier()

    _edge_chunks(zc.at[c], sacc, sidx, didx, (rows_a, rows_b), gsem, ssem)
    plsc.subcore_barrier()

    pltpu.sync_copy(sacc.at[pl.ds(base, RPT)], rows_a.at[pl.ds(0, RPT)])

    def erow(g, _):
        for j in range(8):
            i = g * 8 + j
            rows_b[i, :] = rows_a[i, :] * dv[i, :]
        return 0

    lax.fori_loop(0, RPT // 8, erow, 0)
    pltpu.sync_copy(rows_b.at[pl.ds(0, RPT)], pout.at[c, pl.ds(base, RPT)])


def _edge2_body(
    src_hbm, dst_hbm, dinvb_hbm, p1_hbm, b1_hbm, zc, tout,
    sidx, didx, rows_a, rows_b, dv, zv, b1v, sacc, gsem, ssem,
):
    c = lax.axis_index("c")
    s = lax.axis_index("s")
    w = c * NS + s
    base = s * RPT

    si_cp, di_cp = _edge_pipeline(
        src_hbm, dst_hbm, zc, sacc, sidx, didx, None, gsem, ssem, c, w
    )
    pltpu.sync_copy(dinvb_hbm.at[c, pl.ds(base, RPT)], dv)
    pltpu.sync_copy(p1_hbm.at[0, pl.ds(base, RPT)], rows_a.at[pl.ds(0, RPT)])
    pltpu.sync_copy(p1_hbm.at[1, pl.ds(base, RPT)], rows_b.at[pl.ds(0, RPT)])
    pltpu.sync_copy(b1_hbm, b1v)
    sel = jnp.where(c == 0, 1.0, 0.0).astype(jnp.float32)
    bvec = b1v[...]

    def prow(g, _):
        for j in range(8):
            i = g * 8 + j
            pre = rows_a[i, :] + rows_b[i, :] + bvec
            z = dv[i, :] * jnp.maximum(pre, 0.0)
            zv[i, :] = z
            rows_b[i, :] = z * sel
        return 0

    lax.fori_loop(0, RPT // 8, prow, 0)
    pltpu.sync_copy(zv, zc.at[c, pl.ds(base, RPT)])
    pltpu.sync_copy(rows_b.at[pl.ds(0, RPT)], sacc.at[pl.ds(base, RPT)])
    si_cp.wait()
    di_cp.wait()
    plsc.subcore_barrier()

    _edge_chunks(zc.at[c], sacc, sidx, didx, (rows_a, rows_b), gsem, ssem)
    plsc.subcore_barrier()

    pltpu.sync_copy(sacc.at[pl.ds(base, RPT)], rows_a.at[pl.ds(0, RPT)])

    def erow(g, _):
        for j in range(8):
            i = g * 8 + j
            rows_b[i, :] = rows_a[i, :] * dv[i, :]
        return 0

    lax.fori_loop(0, RPT // 8, erow, 0)
    pltpu.sync_copy(rows_b.at[pl.ds(0, RPT)], tout.at[c, pl.ds(base, RPT)])


_SC_PARAMS = pltpu.CompilerParams(use_tc_tiling_on_sc=False)

_sc_deg = pl.kernel(
    _sc_deg_body,
    out_type=jax.ShapeDtypeStruct((NC, NP), jnp.float32),
    mesh=_MESH,
    scratch_types=[
        pltpu.VMEM((NCHUNK, CH), jnp.int32),
        pltpu.VMEM((CH,), jnp.float32),
        pltpu.VMEM((RPT,), jnp.float32),
        pltpu.VMEM_SHARED((NP,), jnp.float32),
        pltpu.SemaphoreType.DMA,
    ],
    compiler_params=_SC_PARAMS,
)

_sc_dinv = pl.kernel(
    _sc_dinv_body,
    out_type=jax.ShapeDtypeStruct((NC, NPF), jnp.float32),
    mesh=_MESH,
    scratch_types=[
        pltpu.VMEM((RPT,), jnp.float32),
        pltpu.VMEM((RPT,), jnp.float32),
        pltpu.VMEM((RPT,), jnp.float32),
        pltpu.VMEM((RPT * D_HID,), jnp.float32),
    ],
    compiler_params=pltpu.CompilerParams(
        use_tc_tiling_on_sc=False, needs_layout_passes=False
    ),
)

_edge_scratch = [
    pltpu.VMEM((NCHUNK, CH), jnp.int32),
    pltpu.VMEM((NCHUNK, CH), jnp.int32),
    pltpu.VMEM((CH, D_HID), jnp.float32),
    pltpu.VMEM((CH, D_HID), jnp.float32),
    pltpu.VMEM((RPT, D_HID), jnp.float32),
    pltpu.VMEM((RPT, D_HID), jnp.float32),
]

_sc_edge1 = pl.kernel(
    _edge1_body,
    out_type=[
        jax.ShapeDtypeStruct((NC, NP, D_HID), jnp.float32),
        jax.ShapeDtypeStruct((NC, NP, D_HID), jnp.float32),
    ],
    mesh=_MESH,
    scratch_types=_edge_scratch
    + [
        pltpu.VMEM_SHARED((NP, D_HID), jnp.float32),
        pltpu.SemaphoreType.DMA((2,)),
        pltpu.SemaphoreType.DMA((2,)),
    ],
    compiler_params=_SC_PARAMS,
)

_sc_edge2 = pl.kernel(
    _edge2_body,
    out_type=[
        jax.ShapeDtypeStruct((NC, NP, D_HID), jnp.float32),
        jax.ShapeDtypeStruct((NC, NP, D_HID), jnp.float32),
    ],
    mesh=_MESH,
    scratch_types=_edge_scratch
    + [
        pltpu.VMEM((D_HID,), jnp.float32),
        pltpu.VMEM_SHARED((NP, D_HID), jnp.float32),
        pltpu.SemaphoreType.DMA((2,)),
        pltpu.SemaphoreType.DMA((2,)),
    ],
    compiler_params=_SC_PARAMS,
)


def _tc_mm1_body(x_ref, w_ref, y_ref):
    y_ref[...] = jnp.dot(x_ref[...], w_ref[...], preferred_element_type=jnp.float32)


def _tc_out_body(t_ref, w2_ref, b2_ref, o_ref):
    a2 = t_ref[0, :N_NODES, :] + t_ref[1, :N_NODES, :]
    o_ref[...] = (
        jnp.dot(a2, w2_ref[...], preferred_element_type=jnp.float32) + b2_ref[...]
    )


_tc_mm1 = pl.pallas_call(
    _tc_mm1_body, out_shape=jax.ShapeDtypeStruct((N_NODES, D_HID), jnp.float32)
)
_tc_out = pl.pallas_call(
    _tc_out_body, out_shape=jax.ShapeDtypeStruct((N_NODES, D_OUT), jnp.float32)
)


@jax.jit
def kernel(V, E, X, W1, b1, W2, b2):
    Eflat = E.reshape(2 * N_EDGES)
    src2 = lax.slice(Eflat, (0,), (N_EDGES,)).reshape(NW * NCHUNK, CH)
    dst2 = lax.slice(Eflat, (N_EDGES,), (2 * N_EDGES,)).reshape(NW * NCHUNK, CH)

    hist = _sc_deg(dst2)
    dinvb = _sc_dinv(hist).reshape(NC, NP, D_HID)
    Y = _tc_mm1(X, W1)
    Yp = jnp.pad(Y, ((0, NP - N_NODES), (0, 0)))
    _Z1c, P1 = _sc_edge1(src2, dst2, dinvb, Yp)
    _Z2c, T = _sc_edge2(src2, dst2, dinvb, P1, b1)
    return _tc_out(T, W2, b2.reshape(1, D_OUT))


# speedup vs baseline: 89.6543x; 1.1521x over previous
import jax
import jax.numpy as jnp
from jax import lax
from jax.experimental import pallas as pl
from jax.experimental.pallas import tpu as pltpu
from jax.experimental.pallas import tpu_sc as plsc

N_NODES = 10000
N_EDGES = 320000
D_IN = 128
D_HID = 16
D_OUT = 64

NC = 2
NS = 16
NW = NC * NS

NP = 10240
NPF = NP * D_HID
RPT = NP // NS
EPT = N_EDGES // NW
CH = 2000
NCHUNK = EPT // CH

_MESH = plsc.VectorSubcoreMesh(
    core_axis_name="c", subcore_axis_name="s", num_cores=NC, num_subcores=NS
)


def _sc_deg_body(dst_hbm, outd, didx, ones_v, zb, sdeg, sem):
    c = lax.axis_index("c")
    s = lax.axis_index("s")
    w = c * NS + s

    idx_cp = pltpu.async_copy(dst_hbm.at[pl.ds(w * NCHUNK, NCHUNK)], didx, sem)

    def fill(i, _):
        ones_v[pl.ds(i * 16, 16)] = jnp.full((16,), 1.0, jnp.float32)
        return 0

    lax.fori_loop(0, CH // 16, fill, 0)

    def zfill(i, _):
        zb[pl.ds(i * 16, 16)] = jnp.zeros((16,), jnp.float32)
        return 0

    lax.fori_loop(0, RPT // 16, zfill, 0)
    pltpu.sync_copy(zb, sdeg.at[pl.ds(s * RPT, RPT)])
    idx_cp.wait()
    plsc.subcore_barrier()

    scats = []
    for k in range(NCHUNK):
        scats.append(pltpu.async_copy(ones_v, sdeg.at[didx.at[k]], sem, add=True))
    for cp in scats:
        cp.wait()
    plsc.subcore_barrier()
    pltpu.sync_copy(sdeg.at[pl.ds(s * RPT, RPT)], outd.at[c, pl.ds(s * RPT, RPT)])


def _sc_dinv_body(hist_hbm, outd, h0v, h1v, dinvv, dbv):
    c = lax.axis_index("c")
    s = lax.axis_index("s")
    base = s * RPT

    pltpu.sync_copy(hist_hbm.at[0, pl.ds(base, RPT)], h0v)
    pltpu.sync_copy(hist_hbm.at[1, pl.ds(base, RPT)], h1v)

    def newt(g, _):
        d = h0v[pl.ds(g * 16, 16)] + h1v[pl.ds(g * 16, 16)] + 1.0
        i = plsc.bitcast(d, jnp.int32)
        i = jnp.int32(0x5F3759DF) - lax.shift_right_logical(i, 1)
        y = plsc.bitcast(i, jnp.float32)
        for _ in range(3):
            y = y * (1.5 - 0.5 * d * y * y)
        dinvv[pl.ds(g * 16, 16)] = y
        return 0

    lax.fori_loop(0, RPT // 16, newt, 0)

    def splat(i, _):
        dbv[pl.ds(i * 16, 16)] = plsc.load_gather(
            dinvv, [jnp.full((16,), i, jnp.int32)]
        )
        return 0

    lax.fori_loop(0, RPT, splat, 0)
    pltpu.sync_copy(dbv, outd.at[c, pl.ds(base * D_HID, RPT * D_HID)])


def _edge_pipeline(src_hbm, dst_hbm, zc, sacc, sidx, didx, rows, gsem, ssem, c, w):
    si_cp = pltpu.async_copy(src_hbm.at[pl.ds(w * NCHUNK, NCHUNK)], sidx, gsem.at[0])
    di_cp = pltpu.async_copy(dst_hbm.at[pl.ds(w * NCHUNK, NCHUNK)], didx, gsem.at[1])
    return si_cp, di_cp


def _edge_chunks(zc_core, sacc, sidx, didx, rows, gsem, ssem):
    gathers = [None, None]
    scats = [None, None]
    gathers[0] = pltpu.async_copy(zc_core.at[sidx.at[0]], rows[0], gsem.at[0])
    for k in range(NCHUNK):
        cur = k % 2
        nxt = 1 - cur
        gathers[cur].wait()
        if k + 1 < NCHUNK:
            if scats[nxt] is not None:
                scats[nxt].wait()
            gathers[nxt] = pltpu.async_copy(
                zc_core.at[sidx.at[k + 1]], rows[nxt], gsem.at[nxt]
            )
        scats[cur] = pltpu.async_copy(
            rows[cur], sacc.at[didx.at[k]], ssem.at[cur], add=True
        )
    for cp in scats:
        if cp is not None:
            cp.wait()


def _edge1_body(
    src_hbm, dst_hbm, dinvb_hbm, y_hbm, zc, pout,
    sidx, didx, rows_a, rows_b, dv, zv, sacc, gsem, ssem,
):
    c = lax.axis_index("c")
    s = lax.axis_index("s")
    w = c * NS + s
    base = s * RPT

    si_cp, di_cp = _edge_pipeline(
        src_hbm, dst_hbm, zc, sacc, sidx, didx, None, gsem, ssem, c, w
    )
    pltpu.sync_copy(dinvb_hbm.at[c, pl.ds(base, RPT)], dv)
    pltpu.sync_copy(y_hbm.at[pl.ds(base, RPT)], rows_a.at[pl.ds(0, RPT)])
    sel = jnp.where(c == 0, 1.0, 0.0).astype(jnp.float32)

    def prow(g, _):
        for j in range(8):
            i = g * 8 + j
            z = rows_a[i, :] * dv[i, :]
            zv[i, :] = z
            rows_b[i, :] = z * sel
        return 0

    lax.fori_loop(0, RPT // 8, prow, 0)
    pltpu.sync_copy(zv, zc.at[c, pl.ds(base, RPT)])
    pltpu.sync_copy(rows_b.at[pl.ds(0, RPT)], sacc.at[pl.ds(base, RPT)])
    si_cp.wait()
    di_cp.wait()
    plsc.subcore_barrier()

    _edge_chunks(zc.at[c], sacc, sidx, didx, (rows_a, rows_b), gsem, ssem)
    plsc.subcore_barrier()

    pltpu.sync_copy(sacc.at[pl.ds(base, RPT)], rows_a.at[pl.ds(0, RPT)])

    def erow(g, _):
        for j in range(8):
            i = g * 8 + j
            rows_b[i, :] = rows_a[i, :] * dv[i, :]
        return 0

    lax.fori_loop(0, RPT // 8, erow, 0)
    pltpu.sync_copy(rows_b.at[pl.ds(0, RPT)], pout.at[c, pl.ds(base, RPT)])


def _edge2_body(
    src_hbm, dst_hbm, dinvb_hbm, p1_hbm, b1_hbm, zc, tout,
    sidx, didx, rows_a, rows_b, dv, zv, b1v, sacc, gsem, ssem,
):
    c = lax.axis_index("c")
    s = lax.axis_index("s")
    w = c * NS + s
    base = s * RPT

    si_cp, di_cp = _edge_pipeline(
        src_hbm, dst_hbm, zc, sacc, sidx, didx, None, gsem, ssem, c, w
    )
    pltpu.sync_copy(dinvb_hbm.at[c, pl.ds(base, RPT)], dv)
    pltpu.sync_copy(p1_hbm.at[0, pl.ds(base, RPT)], rows_a.at[pl.ds(0, RPT)])
    pltpu.sync_copy(p1_hbm.at[1, pl.ds(base, RPT)], rows_b.at[pl.ds(0, RPT)])
    pltpu.sync_copy(b1_hbm, b1v)
    sel = jnp.where(c == 0, 1.0, 0.0).astype(jnp.float32)
    bvec = b1v[...]

    def prow(g, _):
        for j in range(8):
            i = g * 8 + j
            pre = rows_a[i, :] + rows_b[i, :] + bvec
            z = dv[i, :] * jnp.maximum(pre, 0.0)
            zv[i, :] = z
            rows_b[i, :] = z * sel
        return 0

    lax.fori_loop(0, RPT // 8, prow, 0)
    pltpu.sync_copy(zv, zc.at[c, pl.ds(base, RPT)])
    pltpu.sync_copy(rows_b.at[pl.ds(0, RPT)], sacc.at[pl.ds(base, RPT)])
    si_cp.wait()
    di_cp.wait()
    plsc.subcore_barrier()

    _edge_chunks(zc.at[c], sacc, sidx, didx, (rows_a, rows_b), gsem, ssem)
    plsc.subcore_barrier()

    pltpu.sync_copy(sacc.at[pl.ds(base, RPT)], rows_a.at[pl.ds(0, RPT)])

    def erow(g, _):
        for j in range(8):
            i = g * 8 + j
            rows_b[i, :] = rows_a[i, :] * dv[i, :]
        return 0

    lax.fori_loop(0, RPT // 8, erow, 0)
    pltpu.sync_copy(rows_b.at[pl.ds(0, RPT)], tout.at[c, pl.ds(base, RPT)])


_SC_PARAMS = pltpu.CompilerParams(use_tc_tiling_on_sc=False)

_sc_deg = pl.kernel(
    _sc_deg_body,
    out_type=jax.ShapeDtypeStruct((NC, NP), jnp.float32),
    mesh=_MESH,
    scratch_types=[
        pltpu.VMEM((NCHUNK, CH), jnp.int32),
        pltpu.VMEM((CH,), jnp.float32),
        pltpu.VMEM((RPT,), jnp.float32),
        pltpu.VMEM_SHARED((NP,), jnp.float32),
        pltpu.SemaphoreType.DMA,
    ],
    compiler_params=_SC_PARAMS,
)

_sc_dinv = pl.kernel(
    _sc_dinv_body,
    out_type=jax.ShapeDtypeStruct((NC, NPF), jnp.float32),
    mesh=_MESH,
    scratch_types=[
        pltpu.VMEM((RPT,), jnp.float32),
        pltpu.VMEM((RPT,), jnp.float32),
        pltpu.VMEM((RPT,), jnp.float32),
        pltpu.VMEM((RPT * D_HID,), jnp.float32),
    ],
    compiler_params=pltpu.CompilerParams(
        use_tc_tiling_on_sc=False, needs_layout_passes=False
    ),
)

_edge_scratch = [
    pltpu.VMEM((NCHUNK, CH), jnp.int32),
    pltpu.VMEM((NCHUNK, CH), jnp.int32),
    pltpu.VMEM((CH, D_HID), jnp.float32),
    pltpu.VMEM((CH, D_HID), jnp.float32),
    pltpu.VMEM((RPT, D_HID), jnp.float32),
    pltpu.VMEM((RPT, D_HID), jnp.float32),
]

_sc_edge1 = pl.kernel(
    _edge1_body,
    out_type=[
        jax.ShapeDtypeStruct((NC, NP, D_HID), jnp.float32),
        jax.ShapeDtypeStruct((NC, NP, D_HID), jnp.float32),
    ],
    mesh=_MESH,
    scratch_types=_edge_scratch
    + [
        pltpu.VMEM_SHARED((NP, D_HID), jnp.float32),
        pltpu.SemaphoreType.DMA((2,)),
        pltpu.SemaphoreType.DMA((2,)),
    ],
    compiler_params=_SC_PARAMS,
)

_sc_edge2 = pl.kernel(
    _edge2_body,
    out_type=[
        jax.ShapeDtypeStruct((NC, NP, D_HID), jnp.float32),
        jax.ShapeDtypeStruct((NC, NP, D_HID), jnp.float32),
    ],
    mesh=_MESH,
    scratch_types=_edge_scratch
    + [
        pltpu.VMEM((D_HID,), jnp.float32),
        pltpu.VMEM_SHARED((NP, D_HID), jnp.float32),
        pltpu.SemaphoreType.DMA((2,)),
        pltpu.SemaphoreType.DMA((2,)),
    ],
    compiler_params=_SC_PARAMS,
)


def _tc_mm1_body(x_ref, w_ref, y_ref):
    y_ref[...] = jnp.dot(x_ref[...], w_ref[...], preferred_element_type=jnp.float32)


def _tc_out_body(t_ref, w2_ref, b2_ref, o_ref):
    a2 = t_ref[0, :N_NODES, :] + t_ref[1, :N_NODES, :]
    o_ref[...] = (
        jnp.dot(a2, w2_ref[...], preferred_element_type=jnp.float32) + b2_ref[...]
    )


_tc_mm1 = pl.pallas_call(
    _tc_mm1_body, out_shape=jax.ShapeDtypeStruct((N_NODES, D_HID), jnp.float32)
)
_tc_out = pl.pallas_call(
    _tc_out_body, out_shape=jax.ShapeDtypeStruct((N_NODES, D_OUT), jnp.float32)
)


@jax.jit
def kernel(V, E, X, W1, b1, W2, b2):
    Eflat = lax.optimization_barrier(E.reshape(2 * N_EDGES))
    src2 = lax.slice(Eflat, (0,), (N_EDGES,)).reshape(NW * NCHUNK, CH)
    dst2 = lax.slice(Eflat, (N_EDGES,), (2 * N_EDGES,)).reshape(NW * NCHUNK, CH)

    hist = _sc_deg(dst2)
    dinvb = _sc_dinv(hist).reshape(NC, NP, D_HID)
    Y = _tc_mm1(X, W1)
    Yp = jnp.pad(Y, ((0, NP - N_NODES), (0, 0)))
    _Z1c, P1 = _sc_edge1(src2, dst2, dinvb, Yp)
    _Z2c, T = _sc_edge2(src2, dst2, dinvb, P1, b1)
    return _tc_out(T, W2, b2.reshape(1, D_OUT))
